# Initial kernel scaffold; baseline (speedup 1.0000x reference)
import jax
import jax.numpy as jnp
from jax.experimental import pallas as pl


def kernel(x, emb, W, b):
    raise NotImplementedError("write your pallas kernel here")


# speedup vs baseline: 2.2546x; 2.2546x over previous
import functools

import jax
import jax.numpy as jnp
from jax import lax
from jax.experimental import pallas as pl
from jax.experimental.pallas import tpu as pltpu
from jax.experimental.pallas import tpu_sc as plsc

VOCAB = 100000
DIM = 128
BATCH = 4096
SEQ = 50
N = BATCH * SEQ


ROWS_PER_BLK = 1000


def _linear_body(emb_ref, w_ref, b_ref, out_ref):
    out_ref[...] = lax.dot_general(
        emb_ref[...], w_ref[...],
        dimension_numbers=(((1,), (1,)), ((), ())),
        preferred_element_type=jnp.float32,
    ) + b_ref[...]


def _linear_table(emb, W, b):
    return pl.pallas_call(
        _linear_body,
        grid=(VOCAB // ROWS_PER_BLK,),
        in_specs=[
            pl.BlockSpec((ROWS_PER_BLK, DIM), lambda i: (i, 0)),
            pl.BlockSpec((DIM, DIM), lambda i: (0, 0)),
            pl.BlockSpec((1, DIM), lambda i: (0, 0)),
        ],
        out_specs=pl.BlockSpec((ROWS_PER_BLK, DIM), lambda i: (i, 0)),
        out_shape=jax.ShapeDtypeStruct((VOCAB, DIM), jnp.float32),
    )(emb, W, b.reshape(1, DIM))


CHUNK = 128


@functools.cache
def _make_gather():
    info = plsc.get_sparse_core_info()
    nc, ns = info.num_cores, info.num_subcores
    nw = nc * ns
    b_per_w = N // nw
    n_chunks = b_per_w // CHUNK
    mesh = plsc.VectorSubcoreMesh(core_axis_name="c", subcore_axis_name="s")

    @functools.partial(
        pl.kernel,
        mesh=mesh,
        out_type=jax.ShapeDtypeStruct((nw, n_chunks, CHUNK, DIM), jnp.float32),
        scratch_types=[
            pltpu.VMEM((n_chunks, CHUNK), jnp.int32),
            pltpu.VMEM((CHUNK, DIM), jnp.float32),
            pltpu.VMEM((CHUNK, DIM), jnp.float32),
            pltpu.SemaphoreType.DMA,
            pltpu.SemaphoreType.DMA,
        ],
    )
    def gather(table_hbm, idx_hbm, out_hbm, idx_v, rows0, rows1, sem0, sem1):
        wid = lax.axis_index("s") * nc + lax.axis_index("c")
        pltpu.sync_copy(idx_hbm.at[wid], idx_v)

        def pair_body(p, carry):
            j0 = p * 2
            j1 = j0 + 1
            cp0 = pltpu.async_copy(table_hbm.at[idx_v.at[j0]], rows0, sem0)
            cp1 = pltpu.async_copy(table_hbm.at[idx_v.at[j1]], rows1, sem1)
            cp0.wait()
            w0 = pltpu.async_copy(rows0, out_hbm.at[wid, j0], sem0)
            cp1.wait()
            w1 = pltpu.async_copy(rows1, out_hbm.at[wid, j1], sem1)
            w0.wait()
            w1.wait()
            return carry

        lax.fori_loop(0, n_chunks // 2, pair_body, 0)

    return gather, nw, n_chunks


def kernel(x, emb, W, b):
    table = _linear_table(emb, W, b)
    gather, nw, n_chunks = _make_gather()
    idx = x.astype(jnp.int32).reshape(nw, n_chunks, CHUNK)
    out = gather(table, idx)
    return out.reshape(BATCH, SEQ, DIM)


# speedup vs baseline: 2.2611x; 1.0029x over previous
import functools

import jax
import jax.numpy as jnp
from jax import lax
from jax.experimental import pallas as pl
from jax.experimental.pallas import tpu as pltpu
from jax.experimental.pallas import tpu_sc as plsc

VOCAB = 100000
DIM = 128
BATCH = 4096
SEQ = 50
N = BATCH * SEQ


ROWS_PER_BLK = 1000


def _linear_body(emb_ref, w_ref, b_ref, out_ref):
    out_ref[...] = lax.dot_general(
        emb_ref[...], w_ref[...],
        dimension_numbers=(((1,), (1,)), ((), ())),
        preferred_element_type=jnp.float32,
    ) + b_ref[...]


def _linear_table(emb, W, b):
    return pl.pallas_call(
        _linear_body,
        grid=(VOCAB // ROWS_PER_BLK,),
        in_specs=[
            pl.BlockSpec((ROWS_PER_BLK, DIM), lambda i: (i, 0)),
            pl.BlockSpec((DIM, DIM), lambda i: (0, 0)),
            pl.BlockSpec((1, DIM), lambda i: (0, 0)),
        ],
        out_specs=pl.BlockSpec((ROWS_PER_BLK, DIM), lambda i: (i, 0)),
        out_shape=jax.ShapeDtypeStruct((VOCAB, DIM), jnp.float32),
    )(emb, W, b.reshape(1, DIM))


CHUNK = 128


@functools.cache
def _make_gather():
    info = plsc.get_sparse_core_info()
    nc, ns = info.num_cores, info.num_subcores
    nw = nc * ns
    b_per_w = N // nw
    n_chunks = b_per_w // CHUNK
    mesh = plsc.VectorSubcoreMesh(core_axis_name="c", subcore_axis_name="s")

    @functools.partial(
        pl.kernel,
        mesh=mesh,
        out_type=jax.ShapeDtypeStruct((N, DIM), jnp.float32),
        scratch_types=[
            pltpu.VMEM((n_chunks, CHUNK), jnp.int32),
            pltpu.VMEM((CHUNK, DIM), jnp.float32),
            pltpu.VMEM((CHUNK, DIM), jnp.float32),
            pltpu.SemaphoreType.DMA,
            pltpu.SemaphoreType.DMA,
        ],
    )
    def gather(table_hbm, idx_hbm, out_hbm, idx_v, rows0, rows1, sem0, sem1):
        wid = lax.axis_index("s") * nc + lax.axis_index("c")
        base = wid * b_per_w
        pltpu.sync_copy(idx_hbm.at[wid], idx_v)

        def pair_body(p, carry):
            j0 = p * 2
            j1 = j0 + 1
            cp0 = pltpu.async_copy(table_hbm.at[idx_v.at[j0]], rows0, sem0)
            cp1 = pltpu.async_copy(table_hbm.at[idx_v.at[j1]], rows1, sem1)
            cp0.wait()
            w0 = pltpu.async_copy(
                rows0, out_hbm.at[pl.ds(base + j0 * CHUNK, CHUNK)], sem0)
            cp1.wait()
            w1 = pltpu.async_copy(
                rows1, out_hbm.at[pl.ds(base + j1 * CHUNK, CHUNK)], sem1)
            w0.wait()
            w1.wait()
            return carry

        lax.fori_loop(0, n_chunks // 2, pair_body, 0)

    return gather, nw, n_chunks


def kernel(x, emb, W, b):
    table = _linear_table(emb, W, b)
    gather, nw, n_chunks = _make_gather()
    idx = x.astype(jnp.int32).reshape(nw, n_chunks, CHUNK)
    out = gather(table, idx)
    return out.reshape(BATCH, SEQ, DIM)


# speedup vs baseline: 2.2814x; 1.0090x over previous
import functools

import jax
import jax.numpy as jnp
from jax import lax
from jax.experimental import pallas as pl
from jax.experimental.pallas import tpu as pltpu
from jax.experimental.pallas import tpu_sc as plsc

VOCAB = 100000
DIM = 128
BATCH = 4096
SEQ = 50
N = BATCH * SEQ


ROWS_PER_BLK = 1000


def _linear_body(emb_ref, w_ref, b_ref, out_ref):
    out_ref[...] = lax.dot_general(
        emb_ref[...], w_ref[...],
        dimension_numbers=(((1,), (1,)), ((), ())),
        preferred_element_type=jnp.float32,
    ) + b_ref[...]


def _linear_table(emb, W, b):
    return pl.pallas_call(
        _linear_body,
        grid=(VOCAB // ROWS_PER_BLK,),
        in_specs=[
            pl.BlockSpec((ROWS_PER_BLK, DIM), lambda i: (i, 0)),
            pl.BlockSpec((DIM, DIM), lambda i: (0, 0)),
            pl.BlockSpec((1, DIM), lambda i: (0, 0)),
        ],
        out_specs=pl.BlockSpec((ROWS_PER_BLK, DIM), lambda i: (i, 0)),
        out_shape=jax.ShapeDtypeStruct((VOCAB, DIM), jnp.float32),
    )(emb, W, b.reshape(1, DIM))


CHUNK = 128


@functools.cache
def _make_gather():
    info = plsc.get_sparse_core_info()
    nc, ns = info.num_cores, info.num_subcores
    nw = nc * ns
    b_per_w = N // nw
    n_chunks = b_per_w // CHUNK
    mesh = plsc.VectorSubcoreMesh(core_axis_name="c", subcore_axis_name="s")

    @functools.partial(
        pl.kernel,
        mesh=mesh,
        out_type=jax.ShapeDtypeStruct((N, DIM), jnp.float32),
        scratch_types=[
            pltpu.VMEM((n_chunks, CHUNK), jnp.int32),
            pltpu.VMEM((CHUNK, DIM), jnp.float32),
            pltpu.VMEM((CHUNK, DIM), jnp.float32),
            pltpu.SemaphoreType.DMA,
            pltpu.SemaphoreType.DMA,
        ],
    )
    def gather(table_hbm, idx_hbm, out_hbm, idx_v, rows0, rows1, sem0, sem1):
        wid = lax.axis_index("s") * nc + lax.axis_index("c")
        base = wid * b_per_w
        pltpu.sync_copy(idx_hbm.at[wid], idx_v)

        def pair_body(p, carry):
            j0 = p * 2
            j1 = j0 + 1
            cp0 = pltpu.async_copy(table_hbm.at[idx_v.at[j0]], rows0, sem0)
            cp1 = pltpu.async_copy(table_hbm.at[idx_v.at[j1]], rows1, sem1)
            cp0.wait()
            w0 = pltpu.async_copy(
                rows0, out_hbm.at[pl.ds(base + j0 * CHUNK, CHUNK)], sem0)
            cp1.wait()
            w1 = pltpu.async_copy(
                rows1, out_hbm.at[pl.ds(base + j1 * CHUNK, CHUNK)], sem1)
            w0.wait()
            w1.wait()
            return carry

        lax.fori_loop(0, n_chunks // 2, pair_body, 0)

    return gather, nw, n_chunks


GRP = 32


def _reformat_body(g_ref, out_ref):
    out_ref[...] = g_ref[...].reshape(GRP, SEQ, DIM)


def _reformat(g):
    return pl.pallas_call(
        _reformat_body,
        grid=(BATCH // GRP,),
        in_specs=[pl.BlockSpec((GRP * SEQ, DIM), lambda i: (i, 0))],
        out_specs=pl.BlockSpec((GRP, SEQ, DIM), lambda i: (i, 0, 0)),
        out_shape=jax.ShapeDtypeStruct((BATCH, SEQ, DIM), jnp.float32),
    )(g)


def kernel(x, emb, W, b):
    table = _linear_table(emb, W, b)
    gather, nw, n_chunks = _make_gather()
    idx = x.astype(jnp.int32).reshape(nw, n_chunks, CHUNK)
    out = gather(table, idx)
    return _reformat(out)


# speedup vs baseline: 4.6462x; 2.0366x over previous
import functools

import jax
import jax.numpy as jnp
from jax import lax
from jax.experimental import pallas as pl
from jax.experimental.pallas import tpu as pltpu
from jax.experimental.pallas import tpu_sc as plsc

VOCAB = 100000
DIM = 128
BATCH = 4096
SEQ = 50
N = BATCH * SEQ


ROWS_PER_BLK = 1000


def _linear_body(emb_ref, w_ref, b_ref, out_ref):
    out_ref[...] = lax.dot_general(
        emb_ref[...], w_ref[...],
        dimension_numbers=(((1,), (1,)), ((), ())),
        preferred_element_type=jnp.float32,
    ) + b_ref[...]


def _linear_table(emb, W, b):
    return pl.pallas_call(
        _linear_body,
        grid=(VOCAB // ROWS_PER_BLK,),
        in_specs=[
            pl.BlockSpec((ROWS_PER_BLK, DIM), lambda i: (i, 0)),
            pl.BlockSpec((DIM, DIM), lambda i: (0, 0)),
            pl.BlockSpec((1, DIM), lambda i: (0, 0)),
        ],
        out_specs=pl.BlockSpec((ROWS_PER_BLK, DIM), lambda i: (i, 0)),
        out_shape=jax.ShapeDtypeStruct((VOCAB, DIM), jnp.float32),
    )(emb, W, b.reshape(1, DIM))


CHUNK = 128


@functools.cache
def _make_gather():
    info = plsc.get_sparse_core_info()
    nc, ns = info.num_cores, info.num_subcores
    nw = nc * ns
    b_per_w = N // nw
    n_chunks = b_per_w // CHUNK
    mesh = plsc.VectorSubcoreMesh(core_axis_name="c", subcore_axis_name="s")

    @functools.partial(
        pl.kernel,
        mesh=mesh,
        out_type=jax.ShapeDtypeStruct((N, DIM), jnp.float32),
        scratch_types=[
            pltpu.VMEM((n_chunks, CHUNK), jnp.int32),
            pltpu.VMEM((CHUNK, DIM), jnp.float32),
            pltpu.VMEM((CHUNK, DIM), jnp.float32),
            pltpu.SemaphoreType.DMA,
            pltpu.SemaphoreType.DMA,
        ],
    )
    def gather(table_hbm, idx_hbm, out_hbm, idx_v, rows0, rows1, sem0, sem1):
        wid = lax.axis_index("s") * nc + lax.axis_index("c")
        base = wid * b_per_w
        pltpu.sync_copy(idx_hbm.at[wid], idx_v)

        def pair_body(p, carry):
            j0 = p * 2
            j1 = j0 + 1
            cp0 = pltpu.async_copy(table_hbm.at[idx_v.at[j0]], rows0, sem0)
            cp1 = pltpu.async_copy(table_hbm.at[idx_v.at[j1]], rows1, sem1)
            cp0.wait()
            w0 = pltpu.async_copy(
                rows0, out_hbm.at[pl.ds(base + j0 * CHUNK, CHUNK)], sem0)
            cp1.wait()
            w1 = pltpu.async_copy(
                rows1, out_hbm.at[pl.ds(base + j1 * CHUNK, CHUNK)], sem1)
            w0.wait()
            w1.wait()
            return carry

        lax.fori_loop(0, n_chunks // 2, pair_body, 0)

    return gather, nw, n_chunks


def kernel(x, emb, W, b):
    table = _linear_table(emb, W, b)
    gather, nw, n_chunks = _make_gather()
    idx = x.T.astype(jnp.int32).reshape(nw, n_chunks, CHUNK)
    g_t = gather(table, idx)
    return g_t.reshape(SEQ, BATCH, DIM).transpose(1, 0, 2)


# speedup vs baseline: 6.0781x; 1.3082x over previous
import functools

import jax
import jax.numpy as jnp
from jax import lax
from jax.experimental import pallas as pl
from jax.experimental.pallas import tpu as pltpu
from jax.experimental.pallas import tpu_sc as plsc

VOCAB = 100000
DIM = 128
BATCH = 4096
SEQ = 50
N = BATCH * SEQ


ROWS_PER_BLK = 4000


def _linear_body(emb_ref, w_ref, b_ref, out_ref):
    out_ref[...] = lax.dot_general(
        emb_ref[...], w_ref[...],
        dimension_numbers=(((1,), (1,)), ((), ())),
        preferred_element_type=jnp.float32,
    ) + b_ref[...]


def _linear_table(emb, W, b):
    return pl.pallas_call(
        _linear_body,
        grid=(VOCAB // ROWS_PER_BLK,),
        in_specs=[
            pl.BlockSpec((ROWS_PER_BLK, DIM), lambda i: (i, 0)),
            pl.BlockSpec((DIM, DIM), lambda i: (0, 0)),
            pl.BlockSpec((1, DIM), lambda i: (0, 0)),
        ],
        out_specs=pl.BlockSpec((ROWS_PER_BLK, DIM), lambda i: (i, 0)),
        out_shape=jax.ShapeDtypeStruct((VOCAB, DIM), jnp.float32),
    )(emb, W, b.reshape(1, DIM))


CHUNK = 128


@functools.cache
def _make_gather():
    info = plsc.get_sparse_core_info()
    nc, ns = info.num_cores, info.num_subcores
    nw = nc * ns
    b_per_w = N // nw
    n_chunks = b_per_w // CHUNK
    mesh = plsc.VectorSubcoreMesh(core_axis_name="c", subcore_axis_name="s")

    @functools.partial(
        pl.kernel,
        mesh=mesh,
        out_type=jax.ShapeDtypeStruct((N, DIM), jnp.float32),
        scratch_types=[
            pltpu.VMEM((n_chunks, CHUNK), jnp.int32),
            pltpu.VMEM((CHUNK, DIM), jnp.float32),
            pltpu.VMEM((CHUNK, DIM), jnp.float32),
            pltpu.SemaphoreType.DMA,
            pltpu.SemaphoreType.DMA,
        ],
    )
    def gather(table_hbm, idx_hbm, out_hbm, idx_v, rows0, rows1, sem0, sem1):
        wid = lax.axis_index("s") * nc + lax.axis_index("c")
        base = wid * b_per_w
        pltpu.sync_copy(idx_hbm.at[wid], idx_v)

        def pair_body(p, carry):
            j0 = p * 2
            j1 = j0 + 1
            cp0 = pltpu.async_copy(table_hbm.at[idx_v.at[j0]], rows0, sem0)
            cp1 = pltpu.async_copy(table_hbm.at[idx_v.at[j1]], rows1, sem1)
            cp0.wait()
            w0 = pltpu.async_copy(
                rows0, out_hbm.at[pl.ds(base + j0 * CHUNK, CHUNK)], sem0)
            cp1.wait()
            w1 = pltpu.async_copy(
                rows1, out_hbm.at[pl.ds(base + j1 * CHUNK, CHUNK)], sem1)
            w0.wait()
            w1.wait()
            return carry

        lax.fori_loop(0, n_chunks // 2, pair_body, 0)

    return gather, nw, n_chunks


def kernel(x, emb, W, b):
    table = _linear_table(emb, W, b)
    gather, nw, n_chunks = _make_gather()
    idx = x.T.astype(jnp.int32).reshape(nw, n_chunks, CHUNK)
    g_t = gather(table, idx)
    return g_t.reshape(SEQ, BATCH, DIM).transpose(1, 0, 2)


# speedup vs baseline: 6.3062x; 1.0375x over previous
import functools

import jax
import jax.numpy as jnp
from jax import lax
from jax.experimental import pallas as pl
from jax.experimental.pallas import tpu as pltpu
from jax.experimental.pallas import tpu_sc as plsc

VOCAB = 100000
DIM = 128
BATCH = 4096
SEQ = 50
N = BATCH * SEQ


ROWS_PER_BLK = 10000


def _linear_body(emb_ref, w_ref, b_ref, out_ref):
    out_ref[...] = lax.dot_general(
        emb_ref[...], w_ref[...],
        dimension_numbers=(((1,), (1,)), ((), ())),
        preferred_element_type=jnp.float32,
    ) + b_ref[...]


def _linear_table(emb, W, b):
    return pl.pallas_call(
        _linear_body,
        grid=(VOCAB // ROWS_PER_BLK,),
        in_specs=[
            pl.BlockSpec((ROWS_PER_BLK, DIM), lambda i: (i, 0)),
            pl.BlockSpec((DIM, DIM), lambda i: (0, 0)),
            pl.BlockSpec((1, DIM), lambda i: (0, 0)),
        ],
        out_specs=pl.BlockSpec((ROWS_PER_BLK, DIM), lambda i: (i, 0)),
        out_shape=jax.ShapeDtypeStruct((VOCAB, DIM), jnp.float32),
    )(emb, W, b.reshape(1, DIM))


CHUNK = 128


@functools.cache
def _make_gather():
    info = plsc.get_sparse_core_info()
    nc, ns = info.num_cores, info.num_subcores
    nw = nc * ns
    b_per_w = N // nw
    n_chunks = b_per_w // CHUNK
    mesh = plsc.VectorSubcoreMesh(core_axis_name="c", subcore_axis_name="s")

    @functools.partial(
        pl.kernel,
        mesh=mesh,
        out_type=jax.ShapeDtypeStruct((N, DIM), jnp.float32),
        scratch_types=[
            pltpu.VMEM((n_chunks, CHUNK), jnp.int32),
            pltpu.VMEM((CHUNK, DIM), jnp.float32),
            pltpu.VMEM((CHUNK, DIM), jnp.float32),
            pltpu.SemaphoreType.DMA,
            pltpu.SemaphoreType.DMA,
        ],
    )
    def gather(table_hbm, idx_hbm, out_hbm, idx_v, rows0, rows1, sem0, sem1):
        wid = lax.axis_index("s") * nc + lax.axis_index("c")
        base = wid * b_per_w
        pltpu.sync_copy(idx_hbm.at[wid], idx_v)

        def pair_body(p, carry):
            j0 = p * 2
            j1 = j0 + 1
            cp0 = pltpu.async_copy(table_hbm.at[idx_v.at[j0]], rows0, sem0)
            cp1 = pltpu.async_copy(table_hbm.at[idx_v.at[j1]], rows1, sem1)
            cp0.wait()
            w0 = pltpu.async_copy(
                rows0, out_hbm.at[pl.ds(base + j0 * CHUNK, CHUNK)], sem0)
            cp1.wait()
            w1 = pltpu.async_copy(
                rows1, out_hbm.at[pl.ds(base + j1 * CHUNK, CHUNK)], sem1)
            w0.wait()
            w1.wait()
            return carry

        lax.fori_loop(0, n_chunks // 2, pair_body, 0)

    return gather, nw, n_chunks


def kernel(x, emb, W, b):
    table = _linear_table(emb, W, b)
    gather, nw, n_chunks = _make_gather()
    idx = x.T.astype(jnp.int32).reshape(nw, n_chunks, CHUNK)
    g_t = gather(table, idx)
    return g_t.reshape(SEQ, BATCH, DIM).transpose(1, 0, 2)


# speedup vs baseline: 6.7508x; 1.0705x over previous
import functools

import jax
import jax.numpy as jnp
from jax import lax
from jax.experimental import pallas as pl
from jax.experimental.pallas import tpu as pltpu
from jax.experimental.pallas import tpu_sc as plsc

VOCAB = 100000
DIM = 128
BATCH = 4096
SEQ = 50
N = BATCH * SEQ


ROWS_PER_BLK = 10000


def _linear_body(emb_ref, w_ref, b_ref, out_ref):
    out_ref[...] = lax.dot_general(
        emb_ref[...], w_ref[...],
        dimension_numbers=(((1,), (1,)), ((), ())),
        preferred_element_type=jnp.float32,
    ) + b_ref[...]


def _linear_table(emb, W, b):
    return pl.pallas_call(
        _linear_body,
        grid=(VOCAB // ROWS_PER_BLK,),
        in_specs=[
            pl.BlockSpec((ROWS_PER_BLK, DIM), lambda i: (i, 0)),
            pl.BlockSpec((DIM, DIM), lambda i: (0, 0)),
            pl.BlockSpec((1, DIM), lambda i: (0, 0)),
        ],
        out_specs=pl.BlockSpec((ROWS_PER_BLK, DIM), lambda i: (i, 0)),
        out_shape=jax.ShapeDtypeStruct((VOCAB, DIM), jnp.float32),
    )(emb, W, b.reshape(1, DIM))


CHUNK = 128


@functools.cache
def _make_gather():
    info = plsc.get_sparse_core_info()
    nc, ns = info.num_cores, info.num_subcores
    nw = nc * ns
    b_per_w = N // nw
    n_chunks = b_per_w // CHUNK
    mesh = plsc.VectorSubcoreMesh(core_axis_name="c", subcore_axis_name="s")

    nbuf = 5
    assert n_chunks % nbuf == 0

    @functools.partial(
        pl.kernel,
        mesh=mesh,
        out_type=jax.ShapeDtypeStruct((N, DIM), jnp.float32),
        scratch_types=[
            pltpu.VMEM((n_chunks, CHUNK), jnp.int32),
        ] + [pltpu.VMEM((CHUNK, DIM), jnp.float32) for _ in range(nbuf)]
          + [pltpu.SemaphoreType.DMA for _ in range(2 * nbuf)],
    )
    def gather(table_hbm, idx_hbm, out_hbm, idx_v, *bufs_and_sems):
        rows = bufs_and_sems[:nbuf]
        gsem = bufs_and_sems[nbuf:2 * nbuf]
        wsem = bufs_and_sems[2 * nbuf:]
        wid = lax.axis_index("s") * nc + lax.axis_index("c")
        base = wid * b_per_w
        pltpu.sync_copy(idx_hbm.at[wid], idx_v)

        def start_gather(k, j):
            pltpu.async_copy(table_hbm.at[idx_v.at[j]], rows[k], gsem[k])

        def start_write(k, j):
            pltpu.async_copy(
                rows[k], out_hbm.at[pl.ds(base + j * CHUNK, CHUNK)], wsem[k])

        def wait_gather(k):
            pltpu.make_async_copy(
                table_hbm.at[idx_v.at[0]], rows[k], gsem[k]).wait()

        def wait_write(k):
            pltpu.make_async_copy(
                rows[k], out_hbm.at[pl.ds(base, CHUNK)], wsem[k]).wait()

        for k in range(nbuf):
            start_gather(k, k)

        def body(p, carry):
            j0 = p * nbuf
            for k in range(nbuf):
                wait_gather(k)
                start_write(k, j0 + k)
            for k in range(nbuf):
                wait_write(k)
                start_gather(k, j0 + nbuf + k)
            return carry

        lax.fori_loop(0, n_chunks // nbuf - 1, body, 0)

        j0 = n_chunks - nbuf
        for k in range(nbuf):
            wait_gather(k)
            start_write(k, j0 + k)
        for k in range(nbuf):
            wait_write(k)

    return gather, nw, n_chunks


def kernel(x, emb, W, b):
    table = _linear_table(emb, W, b)
    gather, nw, n_chunks = _make_gather()
    idx = x.T.astype(jnp.int32).reshape(nw, n_chunks, CHUNK)
    g_t = gather(table, idx)
    return g_t.reshape(SEQ, BATCH, DIM).transpose(1, 0, 2)


# speedup vs baseline: 6.8002x; 1.0073x over previous
import functools

import jax
import jax.numpy as jnp
from jax import lax
from jax.experimental import pallas as pl
from jax.experimental.pallas import tpu as pltpu
from jax.experimental.pallas import tpu_sc as plsc

VOCAB = 100000
DIM = 128
BATCH = 4096
SEQ = 50
N = BATCH * SEQ


ROWS_PER_BLK = 10000


def _linear_body(emb_ref, w_ref, b_ref, out_ref):
    out_ref[...] = lax.dot_general(
        emb_ref[...], w_ref[...],
        dimension_numbers=(((1,), (1,)), ((), ())),
        preferred_element_type=jnp.float32,
    ) + b_ref[...]


def _linear_table(emb, W, b):
    return pl.pallas_call(
        _linear_body,
        grid=(VOCAB // ROWS_PER_BLK,),
        in_specs=[
            pl.BlockSpec((ROWS_PER_BLK, DIM), lambda i: (i, 0)),
            pl.BlockSpec((DIM, DIM), lambda i: (0, 0)),
            pl.BlockSpec((1, DIM), lambda i: (0, 0)),
        ],
        out_specs=pl.BlockSpec((ROWS_PER_BLK, DIM), lambda i: (i, 0)),
        out_shape=jax.ShapeDtypeStruct((VOCAB, DIM), jnp.float32),
    )(emb, W, b.reshape(1, DIM))


CHUNK = 128


@functools.cache
def _make_gather():
    info = plsc.get_sparse_core_info()
    nc, ns = info.num_cores, info.num_subcores
    nw = nc * ns
    b_per_w = N // nw
    n_chunks = b_per_w // CHUNK
    mesh = plsc.VectorSubcoreMesh(core_axis_name="c", subcore_axis_name="s")

    nbuf = 7
    n_steady = (n_chunks // nbuf) * nbuf
    n_tail = n_chunks - n_steady

    @functools.partial(
        pl.kernel,
        mesh=mesh,
        out_type=jax.ShapeDtypeStruct((N, DIM), jnp.float32),
        scratch_types=[
            pltpu.VMEM((n_chunks, CHUNK), jnp.int32),
        ] + [pltpu.VMEM((CHUNK, DIM), jnp.float32) for _ in range(nbuf)]
          + [pltpu.SemaphoreType.DMA for _ in range(2 * nbuf)],
    )
    def gather(table_hbm, idx_hbm, out_hbm, idx_v, *bufs_and_sems):
        rows = bufs_and_sems[:nbuf]
        gsem = bufs_and_sems[nbuf:2 * nbuf]
        wsem = bufs_and_sems[2 * nbuf:]
        wid = lax.axis_index("s") * nc + lax.axis_index("c")
        base = wid * b_per_w
        pltpu.sync_copy(idx_hbm.at[wid], idx_v)

        def start_gather(k, j):
            pltpu.async_copy(table_hbm.at[idx_v.at[j]], rows[k], gsem[k])

        def start_write(k, j):
            pltpu.async_copy(
                rows[k], out_hbm.at[pl.ds(base + j * CHUNK, CHUNK)], wsem[k])

        def wait_gather(k):
            pltpu.make_async_copy(
                table_hbm.at[idx_v.at[0]], rows[k], gsem[k]).wait()

        def wait_write(k):
            pltpu.make_async_copy(
                rows[k], out_hbm.at[pl.ds(base, CHUNK)], wsem[k]).wait()

        for k in range(nbuf):
            start_gather(k, k)

        def body(p, carry):
            j0 = p * nbuf
            for k in range(nbuf):
                wait_gather(k)
                start_write(k, j0 + k)
            for k in range(nbuf):
                wait_write(k)
                start_gather(k, j0 + nbuf + k)
            return carry

        lax.fori_loop(0, n_steady // nbuf - 1, body, 0)

        j0 = n_steady - nbuf
        for k in range(nbuf):
            wait_gather(k)
            start_write(k, j0 + k)
        for k in range(n_tail):
            wait_write(k)
            start_gather(k, n_steady + k)
            wait_gather(k)
            start_write(k, n_steady + k)
        for k in range(nbuf):
            wait_write(k)

    return gather, nw, n_chunks


def kernel(x, emb, W, b):
    table = _linear_table(emb, W, b)
    gather, nw, n_chunks = _make_gather()
    idx = x.T.astype(jnp.int32).reshape(nw, n_chunks, CHUNK)
    g_t = gather(table, idx)
    return g_t.reshape(SEQ, BATCH, DIM).transpose(1, 0, 2)


# speedup vs baseline: 6.8343x; 1.0050x over previous
import functools

import jax
import jax.numpy as jnp
from jax import lax
from jax.experimental import pallas as pl
from jax.experimental.pallas import tpu as pltpu
from jax.experimental.pallas import tpu_sc as plsc

VOCAB = 100000
DIM = 128
BATCH = 4096
SEQ = 50
N = BATCH * SEQ


ROWS_PER_BLK = 10000


def _linear_body(emb_ref, w_ref, b_ref, out_ref):
    out_ref[...] = lax.dot_general(
        emb_ref[...], w_ref[...],
        dimension_numbers=(((1,), (1,)), ((), ())),
        preferred_element_type=jnp.float32,
    ) + b_ref[...]


def _linear_table(emb, W, b):
    return pl.pallas_call(
        _linear_body,
        grid=(VOCAB // ROWS_PER_BLK,),
        in_specs=[
            pl.BlockSpec((ROWS_PER_BLK, DIM), lambda i: (i, 0)),
            pl.BlockSpec((DIM, DIM), lambda i: (0, 0)),
            pl.BlockSpec((1, DIM), lambda i: (0, 0)),
        ],
        out_specs=pl.BlockSpec((ROWS_PER_BLK, DIM), lambda i: (i, 0)),
        out_shape=jax.ShapeDtypeStruct((VOCAB, DIM), jnp.float32),
    )(emb, W, b.reshape(1, DIM))


CHUNK = 128


@functools.cache
def _make_gather():
    info = plsc.get_sparse_core_info()
    nc, ns = info.num_cores, info.num_subcores
    nw = nc * ns
    b_per_w = N // nw
    n_chunks = b_per_w // CHUNK
    mesh = plsc.VectorSubcoreMesh(core_axis_name="c", subcore_axis_name="s")

    nbuf = 7
    n_steady = (n_chunks // nbuf) * nbuf
    n_tail = n_chunks - n_steady

    @functools.partial(
        pl.kernel,
        mesh=mesh,
        out_type=jax.ShapeDtypeStruct((N, DIM), jnp.float32),
        scratch_types=[
            pltpu.VMEM((b_per_w,), jnp.int32),
        ] + [pltpu.VMEM((CHUNK, DIM), jnp.float32) for _ in range(nbuf)]
          + [pltpu.SemaphoreType.DMA for _ in range(2 * nbuf)],
    )
    def gather(table_hbm, idx_hbm, out_hbm, idx_v, *bufs_and_sems):
        rows = bufs_and_sems[:nbuf]
        gsem = bufs_and_sems[nbuf:2 * nbuf]
        wsem = bufs_and_sems[2 * nbuf:]
        wid = lax.axis_index("s") * nc + lax.axis_index("c")
        base = wid * b_per_w
        pltpu.sync_copy(idx_hbm.at[pl.ds(base, b_per_w)], idx_v)

        def start_gather(k, j):
            pltpu.async_copy(
                table_hbm.at[idx_v.at[pl.ds(j * CHUNK, CHUNK)]], rows[k],
                gsem[k])

        def start_write(k, j):
            pltpu.async_copy(
                rows[k], out_hbm.at[pl.ds(base + j * CHUNK, CHUNK)], wsem[k])

        def wait_gather(k):
            pltpu.make_async_copy(
                table_hbm.at[idx_v.at[pl.ds(0, CHUNK)]], rows[k],
                gsem[k]).wait()

        def wait_write(k):
            pltpu.make_async_copy(
                rows[k], out_hbm.at[pl.ds(base, CHUNK)], wsem[k]).wait()

        for k in range(nbuf):
            start_gather(k, k)

        def body(p, carry):
            j0 = p * nbuf
            for k in range(nbuf):
                wait_gather(k)
                start_write(k, j0 + k)
            for k in range(nbuf):
                wait_write(k)
                start_gather(k, j0 + nbuf + k)
            return carry

        lax.fori_loop(0, n_steady // nbuf - 1, body, 0)

        j0 = n_steady - nbuf
        for k in range(nbuf):
            wait_gather(k)
            start_write(k, j0 + k)
        for k in range(n_tail):
            wait_write(k)
            start_gather(k, n_steady + k)
            wait_gather(k)
            start_write(k, n_steady + k)
        for k in range(nbuf):
            wait_write(k)

    return gather, nw, n_chunks


def kernel(x, emb, W, b):
    table = _linear_table(emb, W, b)
    gather, nw, n_chunks = _make_gather()
    idx = x.T.astype(jnp.int32).reshape(N)
    g_t = gather(table, idx)
    return g_t.reshape(SEQ, BATCH, DIM).transpose(1, 0, 2)


# speedup vs baseline: 6.9996x; 1.0242x over previous
import functools

import jax
import jax.numpy as jnp
from jax import lax
from jax.experimental import pallas as pl
from jax.experimental.pallas import tpu as pltpu
from jax.experimental.pallas import tpu_sc as plsc

VOCAB = 100000
DIM = 128
BATCH = 4096
SEQ = 50
N = BATCH * SEQ


ROWS_PER_BLK = 10000


def _linear_body(emb_ref, w_ref, b_ref, out_ref):
    out_ref[...] = lax.dot_general(
        emb_ref[...], w_ref[...],
        dimension_numbers=(((1,), (1,)), ((), ())),
        preferred_element_type=jnp.float32,
    ) + b_ref[...]


def _linear_table(emb, W, b):
    return pl.pallas_call(
        _linear_body,
        grid=(VOCAB // ROWS_PER_BLK,),
        in_specs=[
            pl.BlockSpec((ROWS_PER_BLK, DIM), lambda i: (i, 0)),
            pl.BlockSpec((DIM, DIM), lambda i: (0, 0)),
            pl.BlockSpec((1, DIM), lambda i: (0, 0)),
        ],
        out_specs=pl.BlockSpec((ROWS_PER_BLK, DIM), lambda i: (i, 0)),
        out_shape=jax.ShapeDtypeStruct((VOCAB, DIM), jnp.float32),
    )(emb, W, b.reshape(1, DIM))


CHUNK = 128


@functools.cache
def _make_gather():
    info = plsc.get_sparse_core_info()
    nc, ns = info.num_cores, info.num_subcores
    nw = nc * ns
    b_per_w = N // nw
    n_chunks = b_per_w // CHUNK
    mesh = plsc.VectorSubcoreMesh(core_axis_name="c", subcore_axis_name="s")

    nbuf = 4
    nslot = 2
    n_steady = (n_chunks // nbuf) * nbuf
    n_tail = n_chunks - n_steady

    @functools.partial(
        pl.kernel,
        mesh=mesh,
        out_type=jax.ShapeDtypeStruct((N, DIM), jnp.float32),
        scratch_types=[
            pltpu.VMEM((b_per_w,), jnp.int32),
        ] + [pltpu.VMEM((CHUNK, DIM), jnp.float32) for _ in range(nbuf)]
          + [pltpu.VMEM_SHARED((ns, nslot, CHUNK, DIM), jnp.float32)]
          + [pltpu.SemaphoreType.DMA for _ in range(nbuf + nslot)],
    )
    def gather(table_hbm, idx_hbm, out_hbm, idx_v, *rest):
        rows = rest[:nbuf]
        shared = rest[nbuf]
        gsem = rest[nbuf + 1:2 * nbuf + 1]
        wsem = rest[2 * nbuf + 1:]
        sid = lax.axis_index("s")
        wid = sid * nc + lax.axis_index("c")
        base = wid * b_per_w
        pltpu.sync_copy(idx_hbm.at[pl.ds(base, b_per_w)], idx_v)

        def start_gather(k, j):
            pltpu.async_copy(
                table_hbm.at[idx_v.at[pl.ds(j * CHUNK, CHUNK)]], rows[k],
                gsem[k])

        def push_and_write(k, j):
            s = k % nslot
            pltpu.sync_copy(rows[k], shared.at[sid, s])
            pltpu.async_copy(
                shared.at[sid, s],
                out_hbm.at[pl.ds(base + j * CHUNK, CHUNK)], wsem[s])

        def wait_gather(k):
            pltpu.make_async_copy(
                table_hbm.at[idx_v.at[pl.ds(0, CHUNK)]], rows[k],
                gsem[k]).wait()

        def wait_write(s):
            pltpu.make_async_copy(
                shared.at[sid, s], out_hbm.at[pl.ds(base, CHUNK)],
                wsem[s]).wait()

        for k in range(nbuf):
            start_gather(k, k)
        for k in range(nbuf):
            wait_gather(k)
            if k >= nslot:
                wait_write(k % nslot)
            push_and_write(k, k)
            start_gather(k, nbuf + k)

        def body(p, carry):
            j0 = p * nbuf
            for k in range(nbuf):
                wait_gather(k)
                wait_write(k % nslot)
                push_and_write(k, j0 + k)
                start_gather(k, j0 + nbuf + k)
            return carry

        lax.fori_loop(1, n_steady // nbuf - 1, body, 0)

        j0 = n_steady - nbuf
        for k in range(nbuf):
            wait_gather(k)
            wait_write(k % nslot)
            push_and_write(k, j0 + k)
            if k < n_tail:
                start_gather(k, n_steady + k)
        for k in range(n_tail):
            wait_gather(k)
            wait_write(k % nslot)
            push_and_write(k, n_steady + k)
        for s in range(nslot):
            wait_write(s)

    return gather, nw, n_chunks


def kernel(x, emb, W, b):
    table = _linear_table(emb, W, b)
    gather, nw, n_chunks = _make_gather()
    idx = x.T.astype(jnp.int32).reshape(N)
    g_t = gather(table, idx)
    return g_t.reshape(SEQ, BATCH, DIM).transpose(1, 0, 2)


# speedup vs baseline: 7.0722x; 1.0104x over previous
import functools

import jax
import jax.numpy as jnp
from jax import lax
from jax.experimental import pallas as pl
from jax.experimental.pallas import tpu as pltpu
from jax.experimental.pallas import tpu_sc as plsc

VOCAB = 100000
DIM = 128
BATCH = 4096
SEQ = 50
N = BATCH * SEQ


ROWS_PER_BLK = 20000


def _linear_body(emb_ref, w_ref, b_ref, out_ref):
    out_ref[...] = lax.dot_general(
        emb_ref[...], w_ref[...],
        dimension_numbers=(((1,), (1,)), ((), ())),
        preferred_element_type=jnp.float32,
    ) + b_ref[...]


def _linear_table(emb, W, b):
    return pl.pallas_call(
        _linear_body,
        grid=(VOCAB // ROWS_PER_BLK,),
        in_specs=[
            pl.BlockSpec((ROWS_PER_BLK, DIM), lambda i: (i, 0)),
            pl.BlockSpec((DIM, DIM), lambda i: (0, 0)),
            pl.BlockSpec((1, DIM), lambda i: (0, 0)),
        ],
        out_specs=pl.BlockSpec((ROWS_PER_BLK, DIM), lambda i: (i, 0)),
        out_shape=jax.ShapeDtypeStruct((VOCAB, DIM), jnp.float32),
    )(emb, W, b.reshape(1, DIM))


CHUNK = 128


@functools.cache
def _make_gather():
    info = plsc.get_sparse_core_info()
    nc, ns = info.num_cores, info.num_subcores
    nw = nc * ns
    b_per_w = N // nw
    n_chunks = b_per_w // CHUNK
    mesh = plsc.VectorSubcoreMesh(core_axis_name="c", subcore_axis_name="s")

    nbuf = 4
    nslot = 2
    n_steady = (n_chunks // nbuf) * nbuf
    n_tail = n_chunks - n_steady

    @functools.partial(
        pl.kernel,
        mesh=mesh,
        out_type=jax.ShapeDtypeStruct((N, DIM), jnp.float32),
        scratch_types=[
            pltpu.VMEM((b_per_w,), jnp.int32),
        ] + [pltpu.VMEM((CHUNK, DIM), jnp.float32) for _ in range(nbuf)]
          + [pltpu.VMEM_SHARED((ns, nslot, CHUNK, DIM), jnp.float32)]
          + [pltpu.SemaphoreType.DMA for _ in range(nbuf + nslot)],
    )
    def gather(table_hbm, idx_hbm, out_hbm, idx_v, *rest):
        rows = rest[:nbuf]
        shared = rest[nbuf]
        gsem = rest[nbuf + 1:2 * nbuf + 1]
        wsem = rest[2 * nbuf + 1:]
        sid = lax.axis_index("s")
        wid = sid * nc + lax.axis_index("c")
        base = wid * b_per_w
        pltpu.sync_copy(idx_hbm.at[pl.ds(base, b_per_w)], idx_v)

        def start_gather(k, j):
            pltpu.async_copy(
                table_hbm.at[idx_v.at[pl.ds(j * CHUNK, CHUNK)]], rows[k],
                gsem[k])

        def push_and_write(k, j):
            s = k % nslot
            pltpu.sync_copy(rows[k], shared.at[sid, s])
            pltpu.async_copy(
                shared.at[sid, s],
                out_hbm.at[pl.ds(base + j * CHUNK, CHUNK)], wsem[s])

        def wait_gather(k):
            pltpu.make_async_copy(
                table_hbm.at[idx_v.at[pl.ds(0, CHUNK)]], rows[k],
                gsem[k]).wait()

        def wait_write(s):
            pltpu.make_async_copy(
                shared.at[sid, s], out_hbm.at[pl.ds(base, CHUNK)],
                wsem[s]).wait()

        for k in range(nbuf):
            start_gather(k, k)
        for k in range(nbuf):
            wait_gather(k)
            if k >= nslot:
                wait_write(k % nslot)
            push_and_write(k, k)
            start_gather(k, nbuf + k)

        def body(p, carry):
            j0 = p * nbuf
            for k in range(nbuf):
                wait_gather(k)
                wait_write(k % nslot)
                push_and_write(k, j0 + k)
                start_gather(k, j0 + nbuf + k)
            return carry

        lax.fori_loop(1, n_steady // nbuf - 1, body, 0)

        j0 = n_steady - nbuf
        for k in range(nbuf):
            wait_gather(k)
            wait_write(k % nslot)
            push_and_write(k, j0 + k)
            if k < n_tail:
                start_gather(k, n_steady + k)
        for k in range(n_tail):
            wait_gather(k)
            wait_write(k % nslot)
            push_and_write(k, n_steady + k)
        for s in range(nslot):
            wait_write(s)

    return gather, nw, n_chunks


def kernel(x, emb, W, b):
    table = _linear_table(emb, W, b)
    gather, nw, n_chunks = _make_gather()
    idx = x.T.astype(jnp.int32).reshape(N)
    g_t = gather(table, idx)
    return g_t.reshape(SEQ, BATCH, DIM).transpose(1, 0, 2)
